# Initial kernel scaffold; baseline (speedup 1.0000x reference)
import jax
import jax.numpy as jnp
from jax.experimental import pallas as pl


def kernel(x, edge_index, Wl_mu, bl_mu, Wr_mu, Wl_ls, bl_ls, Wr_ls):
    raise NotImplementedError("write your pallas kernel here")


# speedup vs baseline: 9.1949x; 9.1949x over previous
import functools

import jax
import jax.numpy as jnp
from jax import lax
from jax.experimental import pallas as pl
from jax.experimental.pallas import tpu as pltpu
from jax.experimental.pallas import tpu_sc as plsc

N = 10000
D = 128
E = 320000
NC, NS = 2, 16
NW = NC * NS
EPT = E // NW
BATCH = 125
ITERS = EPT // BATCH
IB = 16
NP = ITERS // IB
PE = IB * BATCH
WS = 624
WCH = 104
RB = 1000


def _sc_aggregate(x, e2d, edst):
    mesh = plsc.VectorSubcoreMesh(
        core_axis_name="c", subcore_axis_name="s",
        num_cores=NC, num_subcores=NS)

    @functools.partial(
        pl.kernel,
        out_type=[
            jax.ShapeDtypeStruct((NC, N, D), jnp.float32),
            jax.ShapeDtypeStruct((NW * N,), jnp.float32),
        ],
        mesh=mesh,
        compiler_params=pltpu.CompilerParams(needs_layout_passes=False),
        scratch_types=[
            pltpu.VMEM((IB, BATCH), jnp.int32),
            pltpu.VMEM((IB, BATCH), jnp.int32),
            pltpu.VMEM((PE,), jnp.int32),
            pltpu.VMEM((BATCH, D), jnp.float32),
            pltpu.VMEM((N,), jnp.float32),
            pltpu.VMEM_SHARED((N, D), jnp.float32),
            pltpu.SemaphoreType.DMA,
        ],
    )
    def k(x_hbm, e_hbm, ed_hbm, agg_out, cnt_out,
          src_i, dst_i, dstf, rows, cnt_l, agg_sh, sem):
        c = lax.axis_index("c")
        s = lax.axis_index("s")
        wid = c * NS + s

        zero16 = jnp.zeros((16,), jnp.float32)
        ones16 = jnp.ones((16,), jnp.float32)

        def zrow(i, carry):
            for kk in range(D // 16):
                rows[i, pl.ds(kk * 16, 16)] = zero16
            return carry

        lax.fori_loop(0, BATCH, zrow, 0)

        def zcnt(i, carry):
            cnt_l[pl.ds(i * 16, 16)] = zero16
            return carry

        lax.fori_loop(0, N // 16, zcnt, 0)

        zbase = s * WS
        for kk in range(WS // WCH):
            pltpu.sync_copy(rows.at[pl.ds(0, WCH)],
                            agg_sh.at[pl.ds(zbase + kk * WCH, WCH)])

        @pl.when(s == NS - 1)
        def _ztail():
            pltpu.sync_copy(rows.at[pl.ds(0, N - NS * WS)],
                            agg_sh.at[pl.ds(NS * WS, N - NS * WS)])

        plsc.subcore_barrier()

        for p in range(NP):
            pltpu.sync_copy(e_hbm.at[0, wid, pl.ds(p * IB, IB)], src_i)
            pltpu.sync_copy(e_hbm.at[1, wid, pl.ds(p * IB, IB)], dst_i)
            pltpu.sync_copy(ed_hbm.at[pl.ds(wid * EPT + p * PE, PE)], dstf)

            def edge_body(j, carry):
                pltpu.async_copy(x_hbm.at[src_i.at[j]], rows, sem).wait()
                pltpu.sync_copy(rows, agg_sh.at[dst_i.at[j]], add=True)
                return carry

            lax.fori_loop(0, IB, edge_body, 0)

            def cnt_body(i, carry):
                idx = dstf[pl.ds(i * 16, 16)]
                plsc.addupdate_scatter(cnt_l, [idx], ones16)
                return carry

            lax.fori_loop(0, PE // 16, cnt_body, 0)

        plsc.subcore_barrier()

        for kk in range(WS // WCH):
            off = zbase + kk * WCH
            pltpu.sync_copy(agg_sh.at[pl.ds(off, WCH)], rows.at[pl.ds(0, WCH)])
            pltpu.sync_copy(rows.at[pl.ds(0, WCH)], agg_out.at[c, pl.ds(off, WCH)])

        @pl.when(s == NS - 1)
        def _tail():
            toff = NS * WS
            tn = N - NS * WS
            pltpu.sync_copy(agg_sh.at[pl.ds(toff, tn)], rows.at[pl.ds(0, tn)])
            pltpu.sync_copy(rows.at[pl.ds(0, tn)], agg_out.at[c, pl.ds(toff, tn)])

        pltpu.sync_copy(cnt_l, cnt_out.at[pl.ds(wid * N, N)])

    return k(x, e2d, edst)


def _tc_epilogue(agg_p, cnt_p, x, wlm, blm, wrm, wll, bll, wrl):
    def body(agg_ref, cnt_ref, x_ref, wlm_ref, blm_ref, wrm_ref,
             wll_ref, bll_ref, wrl_ref, mu_ref, ls_ref):
        agg = agg_ref[0] + agg_ref[1]
        deg = jnp.sum(cnt_ref[...], axis=1)[:, None]
        aggn = agg / jnp.maximum(deg, 1.0)
        xb = x_ref[...]
        mu_ref[...] = (
            jnp.dot(aggn, wlm_ref[...], preferred_element_type=jnp.float32)
            + jnp.dot(xb, wrm_ref[...], preferred_element_type=jnp.float32)
            + blm_ref[...])
        ls_ref[...] = (
            jnp.dot(aggn, wll_ref[...], preferred_element_type=jnp.float32)
            + jnp.dot(xb, wrl_ref[...], preferred_element_type=jnp.float32)
            + bll_ref[...])

    grid = (N // RB,)
    wspec = pl.BlockSpec((D, D), lambda i: (0, 0))
    bspec = pl.BlockSpec((1, D), lambda i: (0, 0))
    return pl.pallas_call(
        body,
        grid=grid,
        in_specs=[
            pl.BlockSpec((NC, RB, D), lambda i: (0, i, 0)),
            pl.BlockSpec((RB, NW), lambda i: (i, 0)),
            pl.BlockSpec((RB, D), lambda i: (i, 0)),
            wspec, bspec, wspec, wspec, bspec, wspec,
        ],
        out_specs=[
            pl.BlockSpec((RB, D), lambda i: (i, 0)),
            pl.BlockSpec((RB, D), lambda i: (i, 0)),
        ],
        out_shape=[
            jax.ShapeDtypeStruct((N, D), jnp.float32),
            jax.ShapeDtypeStruct((N, D), jnp.float32),
        ],
    )(agg_p, cnt_p, x, wlm, blm, wrm, wll, bll, wrl)


def kernel(x, edge_index, Wl_mu, bl_mu, Wr_mu, Wl_ls, bl_ls, Wr_ls):
    e2d = edge_index.reshape(2, NW, ITERS, BATCH)
    edst = edge_index[1]
    agg_p, cnt_p = _sc_aggregate(x, e2d, edst)
    mu, logstd = _tc_epilogue(
        agg_p, cnt_p.reshape(NW, N).T, x,
        Wl_mu.T, bl_mu.reshape(1, D), Wr_mu.T,
        Wl_ls.T, bl_ls.reshape(1, D), Wr_ls.T)
    return (mu, logstd)


# speedup vs baseline: 10.7219x; 1.1661x over previous
import functools

import jax
import jax.numpy as jnp
from jax import lax
from jax.experimental import pallas as pl
from jax.experimental.pallas import tpu as pltpu
from jax.experimental.pallas import tpu_sc as plsc

N = 10000
D = 128
E = 320000
NC, NS = 2, 16
NW = NC * NS
EPT = E // NW
BATCH = 125
ITERS = EPT // BATCH
IB = 8
NP = ITERS // IB
CP = 5
PE = EPT // CP
WS = 624
WCH = 104
RB = 1000


def _sc_aggregate(x, e2d, edst):
    mesh = plsc.VectorSubcoreMesh(
        core_axis_name="c", subcore_axis_name="s",
        num_cores=NC, num_subcores=NS)

    @functools.partial(
        pl.kernel,
        out_type=[
            jax.ShapeDtypeStruct((NC, N, D), jnp.float32),
            jax.ShapeDtypeStruct((NW * N,), jnp.float32),
        ],
        mesh=mesh,
        compiler_params=pltpu.CompilerParams(needs_layout_passes=False),
        scratch_types=[
            pltpu.VMEM((IB, BATCH), jnp.int32),
            pltpu.VMEM((IB, BATCH), jnp.int32),
            pltpu.VMEM((PE,), jnp.int32),
            pltpu.VMEM((BATCH, D), jnp.float32),
            pltpu.VMEM((BATCH, D), jnp.float32),
            pltpu.VMEM((N,), jnp.float32),
            pltpu.VMEM_SHARED((N, D), jnp.float32),
            pltpu.SemaphoreType.DMA,
            pltpu.SemaphoreType.DMA,
        ],
    )
    def k(x_hbm, e_hbm, ed_hbm, agg_out, cnt_out,
          src_i, dst_i, dstf, rows, rows1, cnt_l, agg_sh, sem, sem1):
        c = lax.axis_index("c")
        s = lax.axis_index("s")
        wid = c * NS + s

        zero16 = jnp.zeros((16,), jnp.float32)
        ones16 = jnp.ones((16,), jnp.float32)

        def zrow(i, carry):
            for kk in range(D // 16):
                rows[i, pl.ds(kk * 16, 16)] = zero16
            return carry

        lax.fori_loop(0, BATCH, zrow, 0)

        def zcnt(i, carry):
            cnt_l[pl.ds(i * 16, 16)] = zero16
            return carry

        lax.fori_loop(0, N // 16, zcnt, 0)

        zbase = s * WS
        for kk in range(WS // WCH):
            pltpu.sync_copy(rows.at[pl.ds(0, WCH)],
                            agg_sh.at[pl.ds(zbase + kk * WCH, WCH)])

        @pl.when(s == NS - 1)
        def _ztail():
            pltpu.sync_copy(rows.at[pl.ds(0, N - NS * WS)],
                            agg_sh.at[pl.ds(NS * WS, N - NS * WS)])

        plsc.subcore_barrier()

        def g_start(row, buf, sm):
            pltpu.async_copy(x_hbm.at[src_i.at[row]], buf, sm)

        def g_wait(row, buf, sm):
            pltpu.make_async_copy(x_hbm.at[src_i.at[row]], buf, sm).wait()

        def scat(row, buf):
            pltpu.sync_copy(buf, agg_sh.at[dst_i.at[row]], add=True)

        for p in range(NP):
            pltpu.sync_copy(e_hbm.at[0, wid, pl.ds(p * IB, IB)], src_i)
            pltpu.sync_copy(e_hbm.at[1, wid, pl.ds(p * IB, IB)], dst_i)

            g_start(0, rows, sem)

            def dbl(j, carry):
                g_wait(2 * j, rows, sem)
                g_start(2 * j + 1, rows1, sem1)
                scat(2 * j, rows)
                g_wait(2 * j + 1, rows1, sem1)
                g_start(2 * j + 2, rows, sem)
                scat(2 * j + 1, rows1)
                return carry

            lax.fori_loop(0, IB // 2 - 1, dbl, 0)
            g_wait(IB - 2, rows, sem)
            g_start(IB - 1, rows1, sem1)
            scat(IB - 2, rows)
            g_wait(IB - 1, rows1, sem1)
            scat(IB - 1, rows1)

        for q in range(CP):
            pltpu.sync_copy(ed_hbm.at[pl.ds(wid * EPT + q * PE, PE)], dstf)

            def cnt_body(i, carry):
                idx = dstf[pl.ds(i * 16, 16)]
                plsc.addupdate_scatter(cnt_l, [idx], ones16)
                return carry

            lax.fori_loop(0, PE // 16, cnt_body, 0)

        plsc.subcore_barrier()

        for kk in range(WS // WCH):
            off = zbase + kk * WCH
            pltpu.sync_copy(agg_sh.at[pl.ds(off, WCH)], rows.at[pl.ds(0, WCH)])
            pltpu.sync_copy(rows.at[pl.ds(0, WCH)], agg_out.at[c, pl.ds(off, WCH)])

        @pl.when(s == NS - 1)
        def _tail():
            toff = NS * WS
            tn = N - NS * WS
            pltpu.sync_copy(agg_sh.at[pl.ds(toff, tn)], rows.at[pl.ds(0, tn)])
            pltpu.sync_copy(rows.at[pl.ds(0, tn)], agg_out.at[c, pl.ds(toff, tn)])

        pltpu.sync_copy(cnt_l, cnt_out.at[pl.ds(wid * N, N)])

    return k(x, e2d, edst)


def _tc_epilogue(agg_p, cnt_p, x, wlm, blm, wrm, wll, bll, wrl):
    def body(agg_ref, cnt_ref, x_ref, wlm_ref, blm_ref, wrm_ref,
             wll_ref, bll_ref, wrl_ref, mu_ref, ls_ref):
        agg = agg_ref[0] + agg_ref[1]
        deg = jnp.sum(cnt_ref[...], axis=1)[:, None]
        aggn = agg / jnp.maximum(deg, 1.0)
        xb = x_ref[...]
        mu_ref[...] = (
            jnp.dot(aggn, wlm_ref[...], preferred_element_type=jnp.float32)
            + jnp.dot(xb, wrm_ref[...], preferred_element_type=jnp.float32)
            + blm_ref[...])
        ls_ref[...] = (
            jnp.dot(aggn, wll_ref[...], preferred_element_type=jnp.float32)
            + jnp.dot(xb, wrl_ref[...], preferred_element_type=jnp.float32)
            + bll_ref[...])

    grid = (N // RB,)
    wspec = pl.BlockSpec((D, D), lambda i: (0, 0))
    bspec = pl.BlockSpec((1, D), lambda i: (0, 0))
    return pl.pallas_call(
        body,
        grid=grid,
        in_specs=[
            pl.BlockSpec((NC, RB, D), lambda i: (0, i, 0)),
            pl.BlockSpec((RB, NW), lambda i: (i, 0)),
            pl.BlockSpec((RB, D), lambda i: (i, 0)),
            wspec, bspec, wspec, wspec, bspec, wspec,
        ],
        out_specs=[
            pl.BlockSpec((RB, D), lambda i: (i, 0)),
            pl.BlockSpec((RB, D), lambda i: (i, 0)),
        ],
        out_shape=[
            jax.ShapeDtypeStruct((N, D), jnp.float32),
            jax.ShapeDtypeStruct((N, D), jnp.float32),
        ],
    )(agg_p, cnt_p, x, wlm, blm, wrm, wll, bll, wrl)


def kernel(x, edge_index, Wl_mu, bl_mu, Wr_mu, Wl_ls, bl_ls, Wr_ls):
    e2d = edge_index.reshape(2, NW, ITERS, BATCH)
    edst = edge_index[1]
    agg_p, cnt_p = _sc_aggregate(x, e2d, edst)
    mu, logstd = _tc_epilogue(
        agg_p, cnt_p.reshape(NW, N).T, x,
        Wl_mu.T, bl_mu.reshape(1, D), Wr_mu.T,
        Wl_ls.T, bl_ls.reshape(1, D), Wr_ls.T)
    return (mu, logstd)


# speedup vs baseline: 12.1363x; 1.1319x over previous
import functools

import jax
import jax.numpy as jnp
from jax import lax
from jax.experimental import pallas as pl
from jax.experimental.pallas import tpu as pltpu
from jax.experimental.pallas import tpu_sc as plsc

N = 10000
D = 128
E = 320000
NC, NS = 2, 16
NW = NC * NS
EPT = E // NW
BATCH = 125
ITERS = EPT // BATCH
IB = 16
NP = ITERS // IB
WS = 624
WCH = 104
RB = 1000


def _sc_aggregate(x, e2d):
    mesh = plsc.VectorSubcoreMesh(
        core_axis_name="c", subcore_axis_name="s",
        num_cores=NC, num_subcores=NS)

    @functools.partial(
        pl.kernel,
        out_type=[
            jax.ShapeDtypeStruct((NC, N, D), jnp.float32),
            jax.ShapeDtypeStruct((NW * N,), jnp.float32),
        ],
        mesh=mesh,
        compiler_params=pltpu.CompilerParams(needs_layout_passes=False),
        scratch_types=[
            pltpu.VMEM((IB, BATCH), jnp.int32),
            pltpu.VMEM((IB, BATCH), jnp.int32),
            pltpu.VMEM((BATCH, D), jnp.float32),
            pltpu.VMEM((BATCH, D), jnp.float32),
            pltpu.VMEM((N,), jnp.float32),
            pltpu.VMEM_SHARED((N, D), jnp.float32),
            pltpu.SemaphoreType.DMA,
            pltpu.SemaphoreType.DMA,
        ],
    )
    def k(x_hbm, e_hbm, agg_out, cnt_out,
          src_i, dst_i, rows, rows1, cnt_l, agg_sh, sem, sem1):
        c = lax.axis_index("c")
        s = lax.axis_index("s")
        wid = c * NS + s

        zero16 = jnp.zeros((16,), jnp.float32)
        ones16 = jnp.ones((16,), jnp.float32)

        def zrow(i, carry):
            for kk in range(D // 16):
                rows[i, pl.ds(kk * 16, 16)] = zero16
            return carry

        lax.fori_loop(0, BATCH, zrow, 0)

        def zcnt(i, carry):
            cnt_l[pl.ds(i * 16, 16)] = zero16
            return carry

        lax.fori_loop(0, N // 16, zcnt, 0)

        zbase = s * WS
        for kk in range(WS // WCH):
            pltpu.sync_copy(rows.at[pl.ds(0, WCH)],
                            agg_sh.at[pl.ds(zbase + kk * WCH, WCH)])

        @pl.when(s == NS - 1)
        def _ztail():
            pltpu.sync_copy(rows.at[pl.ds(0, N - NS * WS)],
                            agg_sh.at[pl.ds(NS * WS, N - NS * WS)])

        plsc.subcore_barrier()

        def g_start(row, buf, sm):
            pltpu.async_copy(x_hbm.at[src_i.at[row]], buf, sm)

        def g_wait(row, buf, sm):
            pltpu.make_async_copy(x_hbm.at[src_i.at[row]], buf, sm).wait()

        def scat(row, buf):
            pltpu.sync_copy(buf, agg_sh.at[dst_i.at[row]], add=True)

        for p in range(NP):
            pltpu.sync_copy(e_hbm.at[0, wid, pl.ds(p * IB, IB)], src_i)
            pltpu.sync_copy(e_hbm.at[1, wid, pl.ds(p * IB, IB)], dst_i)

            g_start(0, rows, sem)

            def dbl(j, carry):
                g_wait(2 * j, rows, sem)
                g_start(2 * j + 1, rows1, sem1)
                scat(2 * j, rows)
                g_wait(2 * j + 1, rows1, sem1)
                g_start(2 * j + 2, rows, sem)
                scat(2 * j + 1, rows1)
                return carry

            lax.fori_loop(0, IB // 2 - 1, dbl, 0)
            g_wait(IB - 2, rows, sem)
            g_start(IB - 1, rows1, sem1)
            scat(IB - 2, rows)
            g_wait(IB - 1, rows1, sem1)
            scat(IB - 1, rows1)

            tail_mask = lax.iota(jnp.int32, 16) >= 3

            def cnt_body(j, carry):
                for kk in range(7):
                    idx = dst_i[j, pl.ds(kk * 16, 16)]
                    plsc.addupdate_scatter(cnt_l, [idx], ones16)
                idx = dst_i[j, pl.ds(BATCH - 16, 16)]
                plsc.addupdate_scatter(cnt_l, [idx], ones16, mask=tail_mask)
                return carry

            lax.fori_loop(0, IB, cnt_body, 0)

        plsc.subcore_barrier()

        for kk in range(WS // WCH):
            off = zbase + kk * WCH
            pltpu.sync_copy(agg_sh.at[pl.ds(off, WCH)], rows.at[pl.ds(0, WCH)])
            pltpu.sync_copy(rows.at[pl.ds(0, WCH)], agg_out.at[c, pl.ds(off, WCH)])

        @pl.when(s == NS - 1)
        def _tail():
            toff = NS * WS
            tn = N - NS * WS
            pltpu.sync_copy(agg_sh.at[pl.ds(toff, tn)], rows.at[pl.ds(0, tn)])
            pltpu.sync_copy(rows.at[pl.ds(0, tn)], agg_out.at[c, pl.ds(toff, tn)])

        pltpu.sync_copy(cnt_l, cnt_out.at[pl.ds(wid * N, N)])

    return k(x, e2d)


def _tc_epilogue(agg_p, cnt_p, x, wlm, blm, wrm, wll, bll, wrl):
    def body(agg_ref, cnt_ref, x_ref, wlm_ref, blm_ref, wrm_ref,
             wll_ref, bll_ref, wrl_ref, mu_ref, ls_ref):
        agg = agg_ref[0] + agg_ref[1]
        deg = jnp.sum(cnt_ref[...], axis=1)[:, None]
        aggn = agg / jnp.maximum(deg, 1.0)
        xb = x_ref[...]
        mu_ref[...] = (
            jnp.dot(aggn, wlm_ref[...], preferred_element_type=jnp.float32)
            + jnp.dot(xb, wrm_ref[...], preferred_element_type=jnp.float32)
            + blm_ref[...])
        ls_ref[...] = (
            jnp.dot(aggn, wll_ref[...], preferred_element_type=jnp.float32)
            + jnp.dot(xb, wrl_ref[...], preferred_element_type=jnp.float32)
            + bll_ref[...])

    grid = (N // RB,)
    wspec = pl.BlockSpec((D, D), lambda i: (0, 0))
    bspec = pl.BlockSpec((1, D), lambda i: (0, 0))
    return pl.pallas_call(
        body,
        grid=grid,
        in_specs=[
            pl.BlockSpec((NC, RB, D), lambda i: (0, i, 0)),
            pl.BlockSpec((RB, NW), lambda i: (i, 0)),
            pl.BlockSpec((RB, D), lambda i: (i, 0)),
            wspec, bspec, wspec, wspec, bspec, wspec,
        ],
        out_specs=[
            pl.BlockSpec((RB, D), lambda i: (i, 0)),
            pl.BlockSpec((RB, D), lambda i: (i, 0)),
        ],
        out_shape=[
            jax.ShapeDtypeStruct((N, D), jnp.float32),
            jax.ShapeDtypeStruct((N, D), jnp.float32),
        ],
    )(agg_p, cnt_p, x, wlm, blm, wrm, wll, bll, wrl)


def kernel(x, edge_index, Wl_mu, bl_mu, Wr_mu, Wl_ls, bl_ls, Wr_ls):
    e2d = edge_index.reshape(2, NW, ITERS, BATCH)
    agg_p, cnt_p = _sc_aggregate(x, e2d)
    mu, logstd = _tc_epilogue(
        agg_p, cnt_p.reshape(NW, N).T, x,
        Wl_mu.T, bl_mu.reshape(1, D), Wr_mu.T,
        Wl_ls.T, bl_ls.reshape(1, D), Wr_ls.T)
    return (mu, logstd)


# speedup vs baseline: 13.5239x; 1.1143x over previous
import functools

import jax
import jax.numpy as jnp
from jax import lax
from jax.experimental import pallas as pl
from jax.experimental.pallas import tpu as pltpu
from jax.experimental.pallas import tpu_sc as plsc

N = 10000
D = 128
E = 320000
NC, NS = 2, 16
NW = NC * NS
EPT = E // NW
BATCH = 125
ITERS = EPT // BATCH
IB = 16
NP = ITERS // IB
WS = 624
WCH = 104
RB = 1000


def _sc_aggregate(x, e2d):
    mesh = plsc.VectorSubcoreMesh(
        core_axis_name="c", subcore_axis_name="s",
        num_cores=NC, num_subcores=NS)

    @functools.partial(
        pl.kernel,
        out_type=[
            jax.ShapeDtypeStruct((NC, N, D), jnp.float32),
            jax.ShapeDtypeStruct((NW * N,), jnp.float32),
        ],
        mesh=mesh,
        compiler_params=pltpu.CompilerParams(needs_layout_passes=False),
        scratch_types=[
            pltpu.VMEM((IB, BATCH), jnp.int32),
            pltpu.VMEM((IB, BATCH), jnp.int32),
            pltpu.VMEM((BATCH, D), jnp.float32),
            pltpu.VMEM((BATCH, D), jnp.float32),
            pltpu.VMEM((N,), jnp.float32),
            pltpu.VMEM_SHARED((N, D), jnp.float32),
            pltpu.SemaphoreType.DMA,
            pltpu.SemaphoreType.DMA,
        ],
    )
    def k(x_hbm, e_hbm, agg_out, cnt_out,
          src_i, dst_i, rows, rows1, cnt_l, agg_sh, sem, sem1):
        c = lax.axis_index("c")
        s = lax.axis_index("s")
        wid = c * NS + s

        zero16 = jnp.zeros((16,), jnp.float32)
        ones16 = jnp.ones((16,), jnp.float32)

        def zrow(i, carry):
            for kk in range(D // 16):
                rows[i, pl.ds(kk * 16, 16)] = zero16
            return carry

        lax.fori_loop(0, BATCH, zrow, 0)

        def zcnt(i, carry):
            cnt_l[pl.ds(i * 16, 16)] = zero16
            return carry

        lax.fori_loop(0, N // 16, zcnt, 0)

        zbase = s * WS
        for kk in range(WS // WCH):
            pltpu.sync_copy(rows.at[pl.ds(0, WCH)],
                            agg_sh.at[pl.ds(zbase + kk * WCH, WCH)])

        @pl.when(s == NS - 1)
        def _ztail():
            pltpu.sync_copy(rows.at[pl.ds(0, N - NS * WS)],
                            agg_sh.at[pl.ds(NS * WS, N - NS * WS)])

        plsc.subcore_barrier()

        def g_start(row, buf, sm):
            pltpu.async_copy(x_hbm.at[src_i.at[row]], buf, sm)

        def g_wait(row, buf, sm):
            pltpu.make_async_copy(x_hbm.at[src_i.at[row]], buf, sm).wait()

        def scat(row, buf):
            pltpu.sync_copy(buf, agg_sh.at[dst_i.at[row]], add=True)

        for p in range(NP):
            pltpu.sync_copy(e_hbm.at[0, wid, pl.ds(p * IB, IB)], src_i)
            pltpu.sync_copy(e_hbm.at[1, wid, pl.ds(p * IB, IB)], dst_i)

            g_start(0, rows, sem)
            g_start(1, rows1, sem1)

            def dbl(j, carry):
                g_wait(2 * j, rows, sem)
                scat(2 * j, rows)
                g_start(2 * j + 2, rows, sem)
                g_wait(2 * j + 1, rows1, sem1)
                scat(2 * j + 1, rows1)
                g_start(2 * j + 3, rows1, sem1)
                return carry

            lax.fori_loop(0, IB // 2 - 1, dbl, 0)
            g_wait(IB - 2, rows, sem)
            scat(IB - 2, rows)
            g_wait(IB - 1, rows1, sem1)
            scat(IB - 1, rows1)

            tail_mask = lax.iota(jnp.int32, 16) >= 3

            def cnt_body(j, carry):
                for kk in range(7):
                    idx = dst_i[j, pl.ds(kk * 16, 16)]
                    plsc.addupdate_scatter(cnt_l, [idx], ones16)
                idx = dst_i[j, pl.ds(BATCH - 16, 16)]
                plsc.addupdate_scatter(cnt_l, [idx], ones16, mask=tail_mask)
                return carry

            lax.fori_loop(0, IB, cnt_body, 0)

        plsc.subcore_barrier()

        for kk in range(WS // WCH):
            off = zbase + kk * WCH
            pltpu.sync_copy(agg_sh.at[pl.ds(off, WCH)], rows.at[pl.ds(0, WCH)])
            pltpu.sync_copy(rows.at[pl.ds(0, WCH)], agg_out.at[c, pl.ds(off, WCH)])

        @pl.when(s == NS - 1)
        def _tail():
            toff = NS * WS
            tn = N - NS * WS
            pltpu.sync_copy(agg_sh.at[pl.ds(toff, tn)], rows.at[pl.ds(0, tn)])
            pltpu.sync_copy(rows.at[pl.ds(0, tn)], agg_out.at[c, pl.ds(toff, tn)])

        pltpu.sync_copy(cnt_l, cnt_out.at[pl.ds(wid * N, N)])

    return k(x, e2d)


def _tc_epilogue(agg_p, cnt_p, x, wlm, blm, wrm, wll, bll, wrl):
    def body(agg_ref, cnt_ref, x_ref, wlm_ref, blm_ref, wrm_ref,
             wll_ref, bll_ref, wrl_ref, mu_ref, ls_ref):
        agg = agg_ref[0] + agg_ref[1]
        deg = jnp.sum(cnt_ref[...], axis=1)[:, None]
        aggn = agg / jnp.maximum(deg, 1.0)
        xb = x_ref[...]
        mu_ref[...] = (
            jnp.dot(aggn, wlm_ref[...], preferred_element_type=jnp.float32)
            + jnp.dot(xb, wrm_ref[...], preferred_element_type=jnp.float32)
            + blm_ref[...])
        ls_ref[...] = (
            jnp.dot(aggn, wll_ref[...], preferred_element_type=jnp.float32)
            + jnp.dot(xb, wrl_ref[...], preferred_element_type=jnp.float32)
            + bll_ref[...])

    grid = (N // RB,)
    wspec = pl.BlockSpec((D, D), lambda i: (0, 0))
    bspec = pl.BlockSpec((1, D), lambda i: (0, 0))
    return pl.pallas_call(
        body,
        grid=grid,
        in_specs=[
            pl.BlockSpec((NC, RB, D), lambda i: (0, i, 0)),
            pl.BlockSpec((RB, NW), lambda i: (i, 0)),
            pl.BlockSpec((RB, D), lambda i: (i, 0)),
            wspec, bspec, wspec, wspec, bspec, wspec,
        ],
        out_specs=[
            pl.BlockSpec((RB, D), lambda i: (i, 0)),
            pl.BlockSpec((RB, D), lambda i: (i, 0)),
        ],
        out_shape=[
            jax.ShapeDtypeStruct((N, D), jnp.float32),
            jax.ShapeDtypeStruct((N, D), jnp.float32),
        ],
    )(agg_p, cnt_p, x, wlm, blm, wrm, wll, bll, wrl)


def kernel(x, edge_index, Wl_mu, bl_mu, Wr_mu, Wl_ls, bl_ls, Wr_ls):
    e2d = edge_index.reshape(2, NW, ITERS, BATCH)
    agg_p, cnt_p = _sc_aggregate(x, e2d)
    mu, logstd = _tc_epilogue(
        agg_p, cnt_p.reshape(NW, N).T, x,
        Wl_mu.T, bl_mu.reshape(1, D), Wr_mu.T,
        Wl_ls.T, bl_ls.reshape(1, D), Wr_ls.T)
    return (mu, logstd)


# speedup vs baseline: 13.8557x; 1.0245x over previous
import functools

import jax
import jax.numpy as jnp
from jax import lax
from jax.experimental import pallas as pl
from jax.experimental.pallas import tpu as pltpu
from jax.experimental.pallas import tpu_sc as plsc

N = 10000
D = 128
E = 320000
NC, NS = 2, 16
NW = NC * NS
EPT = E // NW
BATCH = 50
ITERS = EPT // BATCH
IB = 40
NP = ITERS // IB
NB = 4
GRP = IB // NB
WS = 624
WCH = 48
RB = 1000


def _sc_aggregate(x, e2d):
    mesh = plsc.VectorSubcoreMesh(
        core_axis_name="c", subcore_axis_name="s",
        num_cores=NC, num_subcores=NS)

    @functools.partial(
        pl.kernel,
        out_type=[
            jax.ShapeDtypeStruct((NC, N, D), jnp.float32),
            jax.ShapeDtypeStruct((NW * N,), jnp.float32),
        ],
        mesh=mesh,
        compiler_params=pltpu.CompilerParams(needs_layout_passes=False),
        scratch_types=[
            pltpu.VMEM((IB, BATCH), jnp.int32),
            pltpu.VMEM((IB, BATCH), jnp.int32),
            pltpu.VMEM((BATCH, D), jnp.float32),
            pltpu.VMEM((BATCH, D), jnp.float32),
            pltpu.VMEM((BATCH, D), jnp.float32),
            pltpu.VMEM((BATCH, D), jnp.float32),
            pltpu.VMEM((N,), jnp.float32),
            pltpu.VMEM_SHARED((N, D), jnp.float32),
            pltpu.SemaphoreType.DMA,
            pltpu.SemaphoreType.DMA,
            pltpu.SemaphoreType.DMA,
            pltpu.SemaphoreType.DMA,
        ],
    )
    def k(x_hbm, e_hbm, agg_out, cnt_out,
          src_i, dst_i, b0, b1, b2, b3, cnt_l, agg_sh, s0, s1, s2, s3):
        c = lax.axis_index("c")
        s = lax.axis_index("s")
        wid = c * NS + s
        bufs = (b0, b1, b2, b3)
        sems = (s0, s1, s2, s3)

        zero16 = jnp.zeros((16,), jnp.float32)
        ones16 = jnp.ones((16,), jnp.float32)

        def zrow(i, carry):
            for kk in range(D // 16):
                b0[i, pl.ds(kk * 16, 16)] = zero16
            return carry

        lax.fori_loop(0, BATCH, zrow, 0)

        def zcnt(i, carry):
            cnt_l[pl.ds(i * 16, 16)] = zero16
            return carry

        lax.fori_loop(0, N // 16, zcnt, 0)

        zbase = s * WS
        for kk in range(WS // WCH):
            pltpu.sync_copy(b0.at[pl.ds(0, WCH)],
                            agg_sh.at[pl.ds(zbase + kk * WCH, WCH)])

        @pl.when(s == NS - 1)
        def _ztail():
            pltpu.sync_copy(b0.at[pl.ds(0, N - NS * WS)],
                            agg_sh.at[pl.ds(NS * WS, N - NS * WS)])

        plsc.subcore_barrier()

        def g_start(row, buf, sm):
            pltpu.async_copy(x_hbm.at[src_i.at[row]], buf, sm)

        def g_wait(row, buf, sm):
            pltpu.make_async_copy(x_hbm.at[src_i.at[row]], buf, sm).wait()

        def scat(row, buf):
            pltpu.sync_copy(buf, agg_sh.at[dst_i.at[row]], add=True)

        tail_mask = lax.iota(jnp.int32, 16) >= 14

        for p in range(NP):
            pltpu.sync_copy(e_hbm.at[0, wid, pl.ds(p * IB, IB)], src_i)
            pltpu.sync_copy(e_hbm.at[1, wid, pl.ds(p * IB, IB)], dst_i)

            for kk in range(NB):
                g_start(kk, bufs[kk], sems[kk])

            def grp_body(j, carry):
                for kk in range(NB):
                    g_wait(NB * j + kk, bufs[kk], sems[kk])
                    scat(NB * j + kk, bufs[kk])
                    g_start(NB * (j + 1) + kk, bufs[kk], sems[kk])
                return carry

            lax.fori_loop(0, GRP - 1, grp_body, 0)
            for kk in range(NB):
                g_wait(IB - NB + kk, bufs[kk], sems[kk])
                scat(IB - NB + kk, bufs[kk])

            def cnt_body(j, carry):
                for kk in range(3):
                    idx = dst_i[j, pl.ds(kk * 16, 16)]
                    plsc.addupdate_scatter(cnt_l, [idx], ones16)
                idx = dst_i[j, pl.ds(BATCH - 16, 16)]
                plsc.addupdate_scatter(cnt_l, [idx], ones16, mask=tail_mask)
                return carry

            lax.fori_loop(0, IB, cnt_body, 0)

        plsc.subcore_barrier()

        for kk in range(WS // WCH):
            off = zbase + kk * WCH
            pltpu.sync_copy(agg_sh.at[pl.ds(off, WCH)], b0.at[pl.ds(0, WCH)])
            pltpu.sync_copy(b0.at[pl.ds(0, WCH)], agg_out.at[c, pl.ds(off, WCH)])

        @pl.when(s == NS - 1)
        def _tail():
            toff = NS * WS
            tn = N - NS * WS
            pltpu.sync_copy(agg_sh.at[pl.ds(toff, tn)], b1.at[pl.ds(0, tn)])
            pltpu.sync_copy(b1.at[pl.ds(0, tn)], agg_out.at[c, pl.ds(toff, tn)])

        pltpu.sync_copy(cnt_l, cnt_out.at[pl.ds(wid * N, N)])

    return k(x, e2d)


def _tc_epilogue(agg_p, cnt_p, x, wlm, blm, wrm, wll, bll, wrl):
    def body(agg_ref, cnt_ref, x_ref, wlm_ref, blm_ref, wrm_ref,
             wll_ref, bll_ref, wrl_ref, mu_ref, ls_ref):
        agg = agg_ref[0] + agg_ref[1]
        deg = jnp.sum(cnt_ref[...], axis=1)[:, None]
        aggn = agg / jnp.maximum(deg, 1.0)
        xb = x_ref[...]
        mu_ref[...] = (
            jnp.dot(aggn, wlm_ref[...], preferred_element_type=jnp.float32)
            + jnp.dot(xb, wrm_ref[...], preferred_element_type=jnp.float32)
            + blm_ref[...])
        ls_ref[...] = (
            jnp.dot(aggn, wll_ref[...], preferred_element_type=jnp.float32)
            + jnp.dot(xb, wrl_ref[...], preferred_element_type=jnp.float32)
            + bll_ref[...])

    grid = (N // RB,)
    wspec = pl.BlockSpec((D, D), lambda i: (0, 0))
    bspec = pl.BlockSpec((1, D), lambda i: (0, 0))
    return pl.pallas_call(
        body,
        grid=grid,
        in_specs=[
            pl.BlockSpec((NC, RB, D), lambda i: (0, i, 0)),
            pl.BlockSpec((RB, NW), lambda i: (i, 0)),
            pl.BlockSpec((RB, D), lambda i: (i, 0)),
            wspec, bspec, wspec, wspec, bspec, wspec,
        ],
        out_specs=[
            pl.BlockSpec((RB, D), lambda i: (i, 0)),
            pl.BlockSpec((RB, D), lambda i: (i, 0)),
        ],
        out_shape=[
            jax.ShapeDtypeStruct((N, D), jnp.float32),
            jax.ShapeDtypeStruct((N, D), jnp.float32),
        ],
    )(agg_p, cnt_p, x, wlm, blm, wrm, wll, bll, wrl)


def kernel(x, edge_index, Wl_mu, bl_mu, Wr_mu, Wl_ls, bl_ls, Wr_ls):
    e2d = edge_index.reshape(2, NW, ITERS, BATCH)
    agg_p, cnt_p = _sc_aggregate(x, e2d)
    mu, logstd = _tc_epilogue(
        agg_p, cnt_p.reshape(NW, N).T, x,
        Wl_mu.T, bl_mu.reshape(1, D), Wr_mu.T,
        Wl_ls.T, bl_ls.reshape(1, D), Wr_ls.T)
    return (mu, logstd)


# speedup vs baseline: 14.0931x; 1.0171x over previous
import functools

import jax
import jax.numpy as jnp
from jax import lax
from jax.experimental import pallas as pl
from jax.experimental.pallas import tpu as pltpu
from jax.experimental.pallas import tpu_sc as plsc

N = 10000
D = 128
E = 320000
NC, NS = 2, 16
NW = NC * NS
EPT = E // NW
BATCH = 50
ITERS = EPT // BATCH
IB = 40
NP = ITERS // IB
NB = 4
GRP = IB // NB
WS = 624
WCH = 48
RB = 1000


def _sc_aggregate(x, e2d):
    mesh = plsc.VectorSubcoreMesh(
        core_axis_name="c", subcore_axis_name="s",
        num_cores=NC, num_subcores=NS)

    @functools.partial(
        pl.kernel,
        out_type=[
            jax.ShapeDtypeStruct((NC, N, D), jnp.float32),
            jax.ShapeDtypeStruct((NW * N,), jnp.float32),
        ],
        mesh=mesh,
        compiler_params=pltpu.CompilerParams(needs_layout_passes=False),
        scratch_types=[
            pltpu.VMEM((IB, BATCH), jnp.int32),
            pltpu.VMEM((IB, BATCH), jnp.int32),
            pltpu.VMEM((BATCH, D), jnp.float32),
            pltpu.VMEM((BATCH, D), jnp.float32),
            pltpu.VMEM((BATCH, D), jnp.float32),
            pltpu.VMEM((BATCH, D), jnp.float32),
            pltpu.VMEM((N,), jnp.float32),
            pltpu.VMEM_SHARED((N, D), jnp.float32),
            pltpu.SemaphoreType.DMA,
            pltpu.SemaphoreType.DMA,
            pltpu.SemaphoreType.DMA,
            pltpu.SemaphoreType.DMA,
        ],
    )
    def k(x_hbm, e_hbm, agg_out, cnt_out,
          src_i, dst_i, b0, b1, b2, b3, cnt_l, agg_sh, s0, s1, s2, s3):
        c = lax.axis_index("c")
        s = lax.axis_index("s")
        wid = c * NS + s
        bufs = (b0, b1, b2, b3)
        sems = (s0, s1, s2, s3)

        zero16 = jnp.zeros((16,), jnp.float32)
        ones16 = jnp.ones((16,), jnp.float32)

        def zrow(i, carry):
            for kk in range(D // 16):
                b0[i, pl.ds(kk * 16, 16)] = zero16
            return carry

        lax.fori_loop(0, BATCH, zrow, 0)

        def zcnt(i, carry):
            cnt_l[pl.ds(i * 16, 16)] = zero16
            return carry

        lax.fori_loop(0, N // 16, zcnt, 0)

        zbase = s * WS
        for kk in range(WS // WCH):
            pltpu.sync_copy(b0.at[pl.ds(0, WCH)],
                            agg_sh.at[pl.ds(zbase + kk * WCH, WCH)])

        @pl.when(s == NS - 1)
        def _ztail():
            pltpu.sync_copy(b0.at[pl.ds(0, N - NS * WS)],
                            agg_sh.at[pl.ds(NS * WS, N - NS * WS)])

        plsc.subcore_barrier()

        def g_start(row, buf, sm):
            pltpu.async_copy(x_hbm.at[src_i.at[row]], buf, sm)

        def g_wait(row, buf, sm):
            pltpu.make_async_copy(x_hbm.at[src_i.at[row]], buf, sm).wait()

        def scat(row, buf):
            pltpu.sync_copy(buf, agg_sh.at[dst_i.at[row]], add=True)

        tail_mask = lax.iota(jnp.int32, 16) >= 14

        for p in range(NP):
            pltpu.sync_copy(e_hbm.at[0, wid, pl.ds(p * IB, IB)], src_i)
            pltpu.sync_copy(e_hbm.at[1, wid, pl.ds(p * IB, IB)], dst_i)

            for kk in range(NB):
                g_start(kk, bufs[kk], sems[kk])

            def grp_body(j, carry):
                for kk in range(NB):
                    g_wait(NB * j + kk, bufs[kk], sems[kk])
                    scat(NB * j + kk, bufs[kk])
                    g_start(NB * (j + 1) + kk, bufs[kk], sems[kk])
                return carry

            lax.fori_loop(0, GRP - 1, grp_body, 0)
            for kk in range(NB):
                g_wait(IB - NB + kk, bufs[kk], sems[kk])
                scat(IB - NB + kk, bufs[kk])

            def cnt_body(j, carry):
                for kk in range(3):
                    idx = dst_i[j, pl.ds(kk * 16, 16)]
                    plsc.addupdate_scatter(cnt_l, [idx], ones16)
                idx = dst_i[j, pl.ds(BATCH - 16, 16)]
                plsc.addupdate_scatter(cnt_l, [idx], ones16, mask=tail_mask)
                return carry

            lax.fori_loop(0, IB, cnt_body, 0)

        pltpu.sync_copy(cnt_l, cnt_out.at[pl.ds(wid * N, N)])
        plsc.subcore_barrier()

        nch = WS // WCH
        wbufs = (b0, b1)
        wsems = (s0, s1)

        def w_hbm(kk, bi):
            pltpu.async_copy(wbufs[bi].at[pl.ds(0, WCH)],
                             agg_out.at[c, pl.ds(zbase + kk * WCH, WCH)],
                             wsems[bi])

        def w_wait(kk, bi):
            pltpu.make_async_copy(
                wbufs[bi].at[pl.ds(0, WCH)],
                agg_out.at[c, pl.ds(zbase + kk * WCH, WCH)],
                wsems[bi]).wait()

        for kk in range(nch):
            bi = kk % 2
            if kk >= 2:
                w_wait(kk - 2, bi)
            pltpu.sync_copy(agg_sh.at[pl.ds(zbase + kk * WCH, WCH)],
                            wbufs[bi].at[pl.ds(0, WCH)])
            w_hbm(kk, bi)
        w_wait(nch - 2, nch % 2)
        w_wait(nch - 1, (nch - 1) % 2)

        @pl.when(s == NS - 1)
        def _tail():
            toff = NS * WS
            tn = N - NS * WS
            pltpu.sync_copy(agg_sh.at[pl.ds(toff, tn)], b2.at[pl.ds(0, tn)])
            pltpu.sync_copy(b2.at[pl.ds(0, tn)], agg_out.at[c, pl.ds(toff, tn)])

    return k(x, e2d)


def _tc_epilogue(agg_p, cnt_p, x, wlm, blm, wrm, wll, bll, wrl):
    def body(agg_ref, cnt_ref, x_ref, wlm_ref, blm_ref, wrm_ref,
             wll_ref, bll_ref, wrl_ref, mu_ref, ls_ref):
        agg = agg_ref[0] + agg_ref[1]
        deg = jnp.sum(cnt_ref[...], axis=1)[:, None]
        aggn = agg / jnp.maximum(deg, 1.0)
        xb = x_ref[...]
        mu_ref[...] = (
            jnp.dot(aggn, wlm_ref[...], preferred_element_type=jnp.float32)
            + jnp.dot(xb, wrm_ref[...], preferred_element_type=jnp.float32)
            + blm_ref[...])
        ls_ref[...] = (
            jnp.dot(aggn, wll_ref[...], preferred_element_type=jnp.float32)
            + jnp.dot(xb, wrl_ref[...], preferred_element_type=jnp.float32)
            + bll_ref[...])

    grid = (N // RB,)
    wspec = pl.BlockSpec((D, D), lambda i: (0, 0))
    bspec = pl.BlockSpec((1, D), lambda i: (0, 0))
    return pl.pallas_call(
        body,
        grid=grid,
        in_specs=[
            pl.BlockSpec((NC, RB, D), lambda i: (0, i, 0)),
            pl.BlockSpec((RB, NW), lambda i: (i, 0)),
            pl.BlockSpec((RB, D), lambda i: (i, 0)),
            wspec, bspec, wspec, wspec, bspec, wspec,
        ],
        out_specs=[
            pl.BlockSpec((RB, D), lambda i: (i, 0)),
            pl.BlockSpec((RB, D), lambda i: (i, 0)),
        ],
        out_shape=[
            jax.ShapeDtypeStruct((N, D), jnp.float32),
            jax.ShapeDtypeStruct((N, D), jnp.float32),
        ],
    )(agg_p, cnt_p, x, wlm, blm, wrm, wll, bll, wrl)


def kernel(x, edge_index, Wl_mu, bl_mu, Wr_mu, Wl_ls, bl_ls, Wr_ls):
    e2d = edge_index.reshape(2, NW, ITERS, BATCH)
    agg_p, cnt_p = _sc_aggregate(x, e2d)
    mu, logstd = _tc_epilogue(
        agg_p, cnt_p.reshape(NW, N).T, x,
        Wl_mu.T, bl_mu.reshape(1, D), Wr_mu.T,
        Wl_ls.T, bl_ls.reshape(1, D), Wr_ls.T)
    return (mu, logstd)


# speedup vs baseline: 14.3117x; 1.0155x over previous
import functools

import jax
import jax.numpy as jnp
from jax import lax
from jax.experimental import pallas as pl
from jax.experimental.pallas import tpu as pltpu
from jax.experimental.pallas import tpu_sc as plsc

N = 10000
D = 128
E = 320000
NC, NS = 2, 16
NW = NC * NS
EPT = E // NW
BATCH = 50
ITERS = EPT // BATCH
IB = 40
NP = ITERS // IB
NB = 4
GRP = IB // NB
WS = 624
WCH = 48
RB = 2000


def _sc_aggregate(x, e2d):
    mesh = plsc.VectorSubcoreMesh(
        core_axis_name="c", subcore_axis_name="s",
        num_cores=NC, num_subcores=NS)

    @functools.partial(
        pl.kernel,
        out_type=[
            jax.ShapeDtypeStruct((NC, N, D), jnp.float32),
            jax.ShapeDtypeStruct((NW * N,), jnp.float32),
        ],
        mesh=mesh,
        compiler_params=pltpu.CompilerParams(needs_layout_passes=False),
        scratch_types=[
            pltpu.VMEM((IB, BATCH), jnp.int32),
            pltpu.VMEM((IB, BATCH), jnp.int32),
            pltpu.VMEM((BATCH, D), jnp.float32),
            pltpu.VMEM((BATCH, D), jnp.float32),
            pltpu.VMEM((BATCH, D), jnp.float32),
            pltpu.VMEM((BATCH, D), jnp.float32),
            pltpu.VMEM((N,), jnp.float32),
            pltpu.VMEM_SHARED((N, D), jnp.float32),
            pltpu.SemaphoreType.DMA,
            pltpu.SemaphoreType.DMA,
            pltpu.SemaphoreType.DMA,
            pltpu.SemaphoreType.DMA,
        ],
    )
    def k(x_hbm, e_hbm, agg_out, cnt_out,
          src_i, dst_i, b0, b1, b2, b3, cnt_l, agg_sh, s0, s1, s2, s3):
        c = lax.axis_index("c")
        s = lax.axis_index("s")
        wid = c * NS + s
        bufs = (b0, b1, b2, b3)
        sems = (s0, s1, s2, s3)

        zero16 = jnp.zeros((16,), jnp.float32)
        ones16 = jnp.ones((16,), jnp.float32)

        def zrow(i, carry):
            for kk in range(D // 16):
                b0[i, pl.ds(kk * 16, 16)] = zero16
            return carry

        lax.fori_loop(0, BATCH, zrow, 0)

        def zcnt(i, carry):
            cnt_l[pl.ds(i * 16, 16)] = zero16
            return carry

        lax.fori_loop(0, N // 16, zcnt, 0)

        zbase = s * WS
        for kk in range(WS // WCH):
            pltpu.sync_copy(b0.at[pl.ds(0, WCH)],
                            agg_sh.at[pl.ds(zbase + kk * WCH, WCH)])

        @pl.when(s == NS - 1)
        def _ztail():
            pltpu.sync_copy(b0.at[pl.ds(0, N - NS * WS)],
                            agg_sh.at[pl.ds(NS * WS, N - NS * WS)])

        plsc.subcore_barrier()

        def g_start(row, buf, sm):
            pltpu.async_copy(x_hbm.at[src_i.at[row]], buf, sm)

        def g_wait(row, buf, sm):
            pltpu.make_async_copy(x_hbm.at[src_i.at[row]], buf, sm).wait()

        def scat(row, buf):
            pltpu.sync_copy(buf, agg_sh.at[dst_i.at[row]], add=True)

        tail_mask = lax.iota(jnp.int32, 16) >= 14

        for p in range(NP):
            pltpu.sync_copy(e_hbm.at[0, wid, pl.ds(p * IB, IB)], src_i)
            pltpu.sync_copy(e_hbm.at[1, wid, pl.ds(p * IB, IB)], dst_i)

            for kk in range(NB):
                g_start(kk, bufs[kk], sems[kk])

            def grp_body(j, carry):
                for kk in range(NB):
                    g_wait(NB * j + kk, bufs[kk], sems[kk])
                    scat(NB * j + kk, bufs[kk])
                    g_start(NB * (j + 1) + kk, bufs[kk], sems[kk])
                return carry

            lax.fori_loop(0, GRP - 1, grp_body, 0)
            for kk in range(NB):
                g_wait(IB - NB + kk, bufs[kk], sems[kk])
                scat(IB - NB + kk, bufs[kk])

            def cnt_body(j, carry):
                for kk in range(3):
                    idx = dst_i[j, pl.ds(kk * 16, 16)]
                    plsc.addupdate_scatter(cnt_l, [idx], ones16)
                idx = dst_i[j, pl.ds(BATCH - 16, 16)]
                plsc.addupdate_scatter(cnt_l, [idx], ones16, mask=tail_mask)
                return carry

            lax.fori_loop(0, IB, cnt_body, 0)

        pltpu.sync_copy(cnt_l, cnt_out.at[pl.ds(wid * N, N)])
        plsc.subcore_barrier()

        nch = WS // WCH
        wbufs = (b0, b1)
        wsems = (s0, s1)

        def w_hbm(kk, bi):
            pltpu.async_copy(wbufs[bi].at[pl.ds(0, WCH)],
                             agg_out.at[c, pl.ds(zbase + kk * WCH, WCH)],
                             wsems[bi])

        def w_wait(kk, bi):
            pltpu.make_async_copy(
                wbufs[bi].at[pl.ds(0, WCH)],
                agg_out.at[c, pl.ds(zbase + kk * WCH, WCH)],
                wsems[bi]).wait()

        for kk in range(nch):
            bi = kk % 2
            if kk >= 2:
                w_wait(kk - 2, bi)
            pltpu.sync_copy(agg_sh.at[pl.ds(zbase + kk * WCH, WCH)],
                            wbufs[bi].at[pl.ds(0, WCH)])
            w_hbm(kk, bi)
        w_wait(nch - 2, nch % 2)
        w_wait(nch - 1, (nch - 1) % 2)

        @pl.when(s == NS - 1)
        def _tail():
            toff = NS * WS
            tn = N - NS * WS
            pltpu.sync_copy(agg_sh.at[pl.ds(toff, tn)], b2.at[pl.ds(0, tn)])
            pltpu.sync_copy(b2.at[pl.ds(0, tn)], agg_out.at[c, pl.ds(toff, tn)])

    return k(x, e2d)


def _tc_epilogue(agg_p, cnt_p, x, wl_cat, wr_cat, b_cat):
    def body(agg_ref, cnt_ref, x_ref, wl_ref, wr_ref, b_ref,
             mu_ref, ls_ref):
        agg = agg_ref[0] + agg_ref[1]
        deg = jnp.sum(cnt_ref[...], axis=1)[:, None]
        aggn = agg / jnp.maximum(deg, 1.0)
        y = (jnp.dot(aggn, wl_ref[...], preferred_element_type=jnp.float32)
             + jnp.dot(x_ref[...], wr_ref[...],
                       preferred_element_type=jnp.float32)
             + b_ref[...])
        mu_ref[...] = y[:, :D]
        ls_ref[...] = y[:, D:]

    grid = (N // RB,)
    return pl.pallas_call(
        body,
        grid=grid,
        in_specs=[
            pl.BlockSpec((NC, RB, D), lambda i: (0, i, 0)),
            pl.BlockSpec((RB, NW), lambda i: (i, 0)),
            pl.BlockSpec((RB, D), lambda i: (i, 0)),
            pl.BlockSpec((D, 2 * D), lambda i: (0, 0)),
            pl.BlockSpec((D, 2 * D), lambda i: (0, 0)),
            pl.BlockSpec((1, 2 * D), lambda i: (0, 0)),
        ],
        out_specs=[
            pl.BlockSpec((RB, D), lambda i: (i, 0)),
            pl.BlockSpec((RB, D), lambda i: (i, 0)),
        ],
        out_shape=[
            jax.ShapeDtypeStruct((N, D), jnp.float32),
            jax.ShapeDtypeStruct((N, D), jnp.float32),
        ],
    )(agg_p, cnt_p, x, wl_cat, wr_cat, b_cat)


def kernel(x, edge_index, Wl_mu, bl_mu, Wr_mu, Wl_ls, bl_ls, Wr_ls):
    e2d = edge_index.reshape(2, NW, ITERS, BATCH)
    agg_p, cnt_p = _sc_aggregate(x, e2d)
    wl_cat = jnp.concatenate([Wl_mu.T, Wl_ls.T], axis=1)
    wr_cat = jnp.concatenate([Wr_mu.T, Wr_ls.T], axis=1)
    b_cat = jnp.concatenate([bl_mu, bl_ls]).reshape(1, 2 * D)
    mu, logstd = _tc_epilogue(agg_p, cnt_p.reshape(NW, N).T, x,
                              wl_cat, wr_cat, b_cat)
    return (mu, logstd)


# speedup vs baseline: 14.6753x; 1.0254x over previous
import functools

import jax
import jax.numpy as jnp
from jax import lax
from jax.experimental import pallas as pl
from jax.experimental.pallas import tpu as pltpu
from jax.experimental.pallas import tpu_sc as plsc

N = 10000
D = 128
E = 320000
NC, NS = 2, 16
NW = NC * NS
EPT = E // NW
BATCH = 50
ITERS = EPT // BATCH
IB = 40
NP = ITERS // IB
NB = 4
GRP = IB // NB
WS = 624
WCH = 48
RB = 2000


def _sc_aggregate(x, e2d):
    mesh = plsc.VectorSubcoreMesh(
        core_axis_name="c", subcore_axis_name="s",
        num_cores=NC, num_subcores=NS)

    @functools.partial(
        pl.kernel,
        out_type=[
            jax.ShapeDtypeStruct((NC, N, D), jnp.float32),
            jax.ShapeDtypeStruct((NW * N,), jnp.float32),
        ],
        mesh=mesh,
        compiler_params=pltpu.CompilerParams(needs_layout_passes=False),
        scratch_types=[
            pltpu.VMEM((IB, BATCH), jnp.int32),
            pltpu.VMEM((IB, BATCH), jnp.int32),
            pltpu.VMEM((BATCH, D), jnp.float32),
            pltpu.VMEM((BATCH, D), jnp.float32),
            pltpu.VMEM((BATCH, D), jnp.float32),
            pltpu.VMEM((BATCH, D), jnp.float32),
            pltpu.VMEM((N,), jnp.float32),
            pltpu.VMEM_SHARED((N, D), jnp.float32),
            pltpu.SemaphoreType.DMA,
            pltpu.SemaphoreType.DMA,
            pltpu.SemaphoreType.DMA,
            pltpu.SemaphoreType.DMA,
        ],
    )
    def k(x_hbm, e_hbm, agg_out, cnt_out,
          src_i, dst_i, b0, b1, b2, b3, cnt_l, agg_sh, s0, s1, s2, s3):
        c = lax.axis_index("c")
        s = lax.axis_index("s")
        wid = c * NS + s
        bufs = (b0, b1, b2, b3)
        sems = (s0, s1, s2, s3)

        zero16 = jnp.zeros((16,), jnp.float32)
        ones16 = jnp.ones((16,), jnp.float32)

        def zrow(i, carry):
            for kk in range(D // 16):
                b0[i, pl.ds(kk * 16, 16)] = zero16
            return carry

        lax.fori_loop(0, BATCH, zrow, 0)

        def zcnt(i, carry):
            cnt_l[pl.ds(i * 16, 16)] = zero16
            return carry

        lax.fori_loop(0, N // 16, zcnt, 0)

        zbase = s * WS
        for kk in range(WS // WCH):
            pltpu.async_copy(b0.at[pl.ds(0, WCH)],
                             agg_sh.at[pl.ds(zbase + kk * WCH, WCH)], s0)
        for kk in range(WS // WCH):
            pltpu.make_async_copy(
                b0.at[pl.ds(0, WCH)],
                agg_sh.at[pl.ds(zbase + kk * WCH, WCH)], s0).wait()

        @pl.when(s == NS - 1)
        def _ztail():
            pltpu.sync_copy(b0.at[pl.ds(0, N - NS * WS)],
                            agg_sh.at[pl.ds(NS * WS, N - NS * WS)])

        plsc.subcore_barrier()

        def g_start(row, buf, sm):
            pltpu.async_copy(x_hbm.at[src_i.at[row]], buf, sm)

        def g_wait(row, buf, sm):
            pltpu.make_async_copy(x_hbm.at[src_i.at[row]], buf, sm).wait()

        def scat(row, buf):
            pltpu.sync_copy(buf, agg_sh.at[dst_i.at[row]], add=True)

        tail_mask = lax.iota(jnp.int32, 16) >= 14

        for p in range(NP):
            pltpu.sync_copy(e_hbm.at[0, wid, pl.ds(p * IB, IB)], src_i)
            pltpu.sync_copy(e_hbm.at[1, wid, pl.ds(p * IB, IB)], dst_i)

            for kk in range(NB):
                g_start(kk, bufs[kk], sems[kk])

            def count_row(row):
                for kk in range(3):
                    idx = dst_i[row, pl.ds(kk * 16, 16)]
                    plsc.addupdate_scatter(cnt_l, [idx], ones16)
                idx = dst_i[row, pl.ds(BATCH - 16, 16)]
                plsc.addupdate_scatter(cnt_l, [idx], ones16, mask=tail_mask)

            def grp_body(j, carry):
                for kk in range(NB):
                    g_wait(NB * j + kk, bufs[kk], sems[kk])
                    scat(NB * j + kk, bufs[kk])
                    g_start(NB * (j + 1) + kk, bufs[kk], sems[kk])
                for kk in range(NB):
                    count_row(NB * j + kk)
                return carry

            lax.fori_loop(0, GRP - 1, grp_body, 0)
            for kk in range(NB):
                g_wait(IB - NB + kk, bufs[kk], sems[kk])
                scat(IB - NB + kk, bufs[kk])
            for kk in range(NB):
                count_row(IB - NB + kk)

        pltpu.sync_copy(cnt_l, cnt_out.at[pl.ds(wid * N, N)])
        plsc.subcore_barrier()

        nch = WS // WCH
        wbufs = (b0, b1)
        wsems = (s0, s1)

        def w_hbm(kk, bi):
            pltpu.async_copy(wbufs[bi].at[pl.ds(0, WCH)],
                             agg_out.at[c, pl.ds(zbase + kk * WCH, WCH)],
                             wsems[bi])

        def w_wait(kk, bi):
            pltpu.make_async_copy(
                wbufs[bi].at[pl.ds(0, WCH)],
                agg_out.at[c, pl.ds(zbase + kk * WCH, WCH)],
                wsems[bi]).wait()

        for kk in range(nch):
            bi = kk % 2
            if kk >= 2:
                w_wait(kk - 2, bi)
            pltpu.sync_copy(agg_sh.at[pl.ds(zbase + kk * WCH, WCH)],
                            wbufs[bi].at[pl.ds(0, WCH)])
            w_hbm(kk, bi)
        w_wait(nch - 2, nch % 2)
        w_wait(nch - 1, (nch - 1) % 2)

        @pl.when(s == NS - 1)
        def _tail():
            toff = NS * WS
            tn = N - NS * WS
            pltpu.sync_copy(agg_sh.at[pl.ds(toff, tn)], b2.at[pl.ds(0, tn)])
            pltpu.sync_copy(b2.at[pl.ds(0, tn)], agg_out.at[c, pl.ds(toff, tn)])

    return k(x, e2d)


def _tc_epilogue(agg_p, cnt_p, x, wl_cat, wr_cat, b_cat):
    def body(agg_ref, cnt_ref, x_ref, wl_ref, wr_ref, b_ref,
             mu_ref, ls_ref):
        agg = agg_ref[0] + agg_ref[1]
        deg = jnp.sum(cnt_ref[...], axis=1)[:, None]
        aggn = agg / jnp.maximum(deg, 1.0)
        y = (jnp.dot(aggn, wl_ref[...], preferred_element_type=jnp.float32)
             + jnp.dot(x_ref[...], wr_ref[...],
                       preferred_element_type=jnp.float32)
             + b_ref[...])
        mu_ref[...] = y[:, :D]
        ls_ref[...] = y[:, D:]

    grid = (N // RB,)
    return pl.pallas_call(
        body,
        grid=grid,
        in_specs=[
            pl.BlockSpec((NC, RB, D), lambda i: (0, i, 0)),
            pl.BlockSpec((RB, NW), lambda i: (i, 0)),
            pl.BlockSpec((RB, D), lambda i: (i, 0)),
            pl.BlockSpec((D, 2 * D), lambda i: (0, 0)),
            pl.BlockSpec((D, 2 * D), lambda i: (0, 0)),
            pl.BlockSpec((1, 2 * D), lambda i: (0, 0)),
        ],
        out_specs=[
            pl.BlockSpec((RB, D), lambda i: (i, 0)),
            pl.BlockSpec((RB, D), lambda i: (i, 0)),
        ],
        out_shape=[
            jax.ShapeDtypeStruct((N, D), jnp.float32),
            jax.ShapeDtypeStruct((N, D), jnp.float32),
        ],
    )(agg_p, cnt_p, x, wl_cat, wr_cat, b_cat)


def kernel(x, edge_index, Wl_mu, bl_mu, Wr_mu, Wl_ls, bl_ls, Wr_ls):
    e2d = edge_index.reshape(2, NW, ITERS, BATCH)
    agg_p, cnt_p = _sc_aggregate(x, e2d)
    wl_cat = jnp.concatenate([Wl_mu.T, Wl_ls.T], axis=1)
    wr_cat = jnp.concatenate([Wr_mu.T, Wr_ls.T], axis=1)
    b_cat = jnp.concatenate([bl_mu, bl_ls]).reshape(1, 2 * D)
    mu, logstd = _tc_epilogue(agg_p, cnt_p.reshape(NW, N).T, x,
                              wl_cat, wr_cat, b_cat)
    return (mu, logstd)


# speedup vs baseline: 15.1231x; 1.0305x over previous
import functools

import jax
import jax.numpy as jnp
from jax import lax
from jax.experimental import pallas as pl
from jax.experimental.pallas import tpu as pltpu
from jax.experimental.pallas import tpu_sc as plsc

N = 10000
D = 128
E = 320000
NC, NS = 2, 16
NW = NC * NS
EPT = E // NW
BATCH = 50
ITERS = EPT // BATCH
IB = 40
NP = ITERS // IB
NB = 4
GRP = IB // NB
WS = 624
WCH = 48
RB = 2000


def _sc_aggregate(x, e2d):
    mesh = plsc.VectorSubcoreMesh(
        core_axis_name="c", subcore_axis_name="s",
        num_cores=NC, num_subcores=NS)

    @functools.partial(
        pl.kernel,
        out_type=[
            jax.ShapeDtypeStruct((NC, N, D), jnp.float32),
            jax.ShapeDtypeStruct((NW * N,), jnp.float32),
        ],
        mesh=mesh,
        compiler_params=pltpu.CompilerParams(needs_layout_passes=False),
        scratch_types=[
            pltpu.VMEM((IB, BATCH), jnp.int32),
            pltpu.VMEM((IB, BATCH), jnp.int32),
            pltpu.VMEM((BATCH, D), jnp.float32),
            pltpu.VMEM((BATCH, D), jnp.float32),
            pltpu.VMEM((BATCH, D), jnp.float32),
            pltpu.VMEM((BATCH, D), jnp.float32),
            pltpu.VMEM((N,), jnp.float32),
            pltpu.VMEM_SHARED((N, D), jnp.float32),
            pltpu.SemaphoreType.DMA,
            pltpu.SemaphoreType.DMA,
            pltpu.SemaphoreType.DMA,
            pltpu.SemaphoreType.DMA,
        ],
    )
    def k(x_hbm, e_hbm, agg_out, cnt_out,
          src_i, dst_i, b0, b1, b2, b3, cnt_l, agg_sh, s0, s1, s2, s3):
        c = lax.axis_index("c")
        s = lax.axis_index("s")
        wid = c * NS + s
        bufs = (b0, b1, b2, b3)
        sems = (s0, s1, s2, s3)

        zero16 = jnp.zeros((16,), jnp.float32)
        ones16 = jnp.ones((16,), jnp.float32)

        def zrow(i, carry):
            for kk in range(D // 16):
                b0[i, pl.ds(kk * 16, 16)] = zero16
            return carry

        lax.fori_loop(0, BATCH, zrow, 0)

        def zcnt(i, carry):
            cnt_l[pl.ds(i * 16, 16)] = zero16
            return carry

        lax.fori_loop(0, N // 16, zcnt, 0)

        pltpu.async_copy(e_hbm.at[0, wid, pl.ds(0, IB)], src_i, s2)
        pltpu.async_copy(e_hbm.at[1, wid, pl.ds(0, IB)], dst_i, s3)

        zbase = s * WS
        for kk in range(WS // WCH):
            pltpu.async_copy(b0.at[pl.ds(0, WCH)],
                             agg_sh.at[pl.ds(zbase + kk * WCH, WCH)], s0)
        for kk in range(WS // WCH):
            pltpu.make_async_copy(
                b0.at[pl.ds(0, WCH)],
                agg_sh.at[pl.ds(zbase + kk * WCH, WCH)], s0).wait()

        @pl.when(s == NS - 1)
        def _ztail():
            pltpu.sync_copy(b0.at[pl.ds(0, N - NS * WS)],
                            agg_sh.at[pl.ds(NS * WS, N - NS * WS)])

        plsc.subcore_barrier()

        def g_start(row, buf, sm):
            pltpu.async_copy(x_hbm.at[src_i.at[row]], buf, sm)

        def g_wait(row, buf, sm):
            pltpu.make_async_copy(x_hbm.at[src_i.at[row]], buf, sm).wait()

        def scat(row, buf):
            pltpu.sync_copy(buf, agg_sh.at[dst_i.at[row]], add=True)

        tail_mask = lax.iota(jnp.int32, 16) >= 14

        def stage(p):
            pltpu.async_copy(e_hbm.at[0, wid, pl.ds(p * IB, IB)], src_i, s2)
            pltpu.async_copy(e_hbm.at[1, wid, pl.ds(p * IB, IB)], dst_i, s3)

        def stage_wait(p):
            pltpu.make_async_copy(
                e_hbm.at[0, wid, pl.ds(p * IB, IB)], src_i, s2).wait()
            pltpu.make_async_copy(
                e_hbm.at[1, wid, pl.ds(p * IB, IB)], dst_i, s3).wait()

        for p in range(NP):
            if p > 0:
                stage(p)
            stage_wait(p)

            for kk in range(NB):
                g_start(kk, bufs[kk], sems[kk])

            def count_row(row):
                for kk in range(3):
                    idx = dst_i[row, pl.ds(kk * 16, 16)]
                    plsc.addupdate_scatter(cnt_l, [idx], ones16)
                idx = dst_i[row, pl.ds(BATCH - 16, 16)]
                plsc.addupdate_scatter(cnt_l, [idx], ones16, mask=tail_mask)

            def grp_body(j, carry):
                for kk in range(NB):
                    g_wait(NB * j + kk, bufs[kk], sems[kk])
                    scat(NB * j + kk, bufs[kk])
                    g_start(NB * (j + 1) + kk, bufs[kk], sems[kk])
                for kk in range(NB):
                    count_row(NB * j + kk)
                return carry

            lax.fori_loop(0, GRP - 1, grp_body, 0)
            for kk in range(NB):
                g_wait(IB - NB + kk, bufs[kk], sems[kk])
                scat(IB - NB + kk, bufs[kk])
            for kk in range(NB):
                count_row(IB - NB + kk)

        pltpu.sync_copy(cnt_l, cnt_out.at[pl.ds(wid * N, N)])
        plsc.subcore_barrier()

        nch = WS // WCH
        wbufs = (b0, b1)
        wsems = (s0, s1)

        def w_hbm(kk, bi):
            pltpu.async_copy(wbufs[bi].at[pl.ds(0, WCH)],
                             agg_out.at[c, pl.ds(zbase + kk * WCH, WCH)],
                             wsems[bi])

        def w_wait(kk, bi):
            pltpu.make_async_copy(
                wbufs[bi].at[pl.ds(0, WCH)],
                agg_out.at[c, pl.ds(zbase + kk * WCH, WCH)],
                wsems[bi]).wait()

        for kk in range(nch):
            bi = kk % 2
            if kk >= 2:
                w_wait(kk - 2, bi)
            pltpu.sync_copy(agg_sh.at[pl.ds(zbase + kk * WCH, WCH)],
                            wbufs[bi].at[pl.ds(0, WCH)])
            w_hbm(kk, bi)
        w_wait(nch - 2, nch % 2)
        w_wait(nch - 1, (nch - 1) % 2)

        @pl.when(s == NS - 1)
        def _tail():
            toff = NS * WS
            tn = N - NS * WS
            pltpu.sync_copy(agg_sh.at[pl.ds(toff, tn)], b2.at[pl.ds(0, tn)])
            pltpu.sync_copy(b2.at[pl.ds(0, tn)], agg_out.at[c, pl.ds(toff, tn)])

    return k(x, e2d)


def _tc_epilogue(agg_p, cnt_p, x, wl_cat, wr_cat, b_cat):
    def body(agg_ref, cnt_ref, x_ref, wl_ref, wr_ref, b_ref,
             mu_ref, ls_ref):
        agg = agg_ref[0] + agg_ref[1]
        deg = jnp.sum(cnt_ref[...], axis=1)[:, None]
        aggn = agg / jnp.maximum(deg, 1.0)
        y = (jnp.dot(aggn, wl_ref[...], preferred_element_type=jnp.float32)
             + jnp.dot(x_ref[...], wr_ref[...],
                       preferred_element_type=jnp.float32)
             + b_ref[...])
        mu_ref[...] = y[:, :D]
        ls_ref[...] = y[:, D:]

    grid = (N // RB,)
    return pl.pallas_call(
        body,
        grid=grid,
        in_specs=[
            pl.BlockSpec((NC, RB, D), lambda i: (0, i, 0)),
            pl.BlockSpec((RB, NW), lambda i: (i, 0)),
            pl.BlockSpec((RB, D), lambda i: (i, 0)),
            pl.BlockSpec((D, 2 * D), lambda i: (0, 0)),
            pl.BlockSpec((D, 2 * D), lambda i: (0, 0)),
            pl.BlockSpec((1, 2 * D), lambda i: (0, 0)),
        ],
        out_specs=[
            pl.BlockSpec((RB, D), lambda i: (i, 0)),
            pl.BlockSpec((RB, D), lambda i: (i, 0)),
        ],
        out_shape=[
            jax.ShapeDtypeStruct((N, D), jnp.float32),
            jax.ShapeDtypeStruct((N, D), jnp.float32),
        ],
    )(agg_p, cnt_p, x, wl_cat, wr_cat, b_cat)


def kernel(x, edge_index, Wl_mu, bl_mu, Wr_mu, Wl_ls, bl_ls, Wr_ls):
    e2d = edge_index.reshape(2, NW, ITERS, BATCH)
    agg_p, cnt_p = _sc_aggregate(x, e2d)
    wl_cat = jnp.concatenate([Wl_mu.T, Wl_ls.T], axis=1)
    wr_cat = jnp.concatenate([Wr_mu.T, Wr_ls.T], axis=1)
    b_cat = jnp.concatenate([bl_mu, bl_ls]).reshape(1, 2 * D)
    mu, logstd = _tc_epilogue(agg_p, cnt_p.reshape(NW, N).T, x,
                              wl_cat, wr_cat, b_cat)
    return (mu, logstd)
